# baseline (speedup 1.0000x reference)
import functools

import jax
import jax.numpy as jnp
import numpy as np
from jax import lax
from jax.experimental import pallas as pl
from jax.experimental.pallas import tpu as pltpu
from jax.experimental.pallas import tpu_sc as plsc

N_NODES = 10000
N_EDGES = 320000
E_TOT = N_EDGES + N_NODES
NW = 32
K_EDGE = 112
STEPS = 94
T_EDGE = STEPS * K_EDGE
E_PAD = NW * T_EDGE
N_ACC = 10000
ROW_BLK = 1000


def _perm(d, heads):
    k = np.arange(d)
    j, l = k // 32, k % 32
    if heads > 1:
        p = (2 * j + (l & 1)) * 16 + (l >> 1)
    else:
        p = 32 * j + (l & 1) * 16 + (l >> 1)
    return p


def _prep_body(x_ref, w_ref, a_ref, hbf_ref, asrc_ref, adst_ref, cvec_ref,
               mx_ref):
    i = pl.program_id(0)
    h = jnp.dot(x_ref[...], w_ref[...], preferred_element_type=jnp.float32)
    ap = jnp.dot(h, a_ref[...], preferred_element_type=jnp.float32)
    hbf_ref[...] = h.astype(jnp.bfloat16)
    asrc_ref[...] = ap[:, 0:16]
    adst_ref[...] = ap[:, 16:32]
    blkmax = jnp.max(ap[:, 0:16], axis=0, keepdims=True)

    @pl.when(i == 0)
    def _():
        mx_ref[...] = blkmax

    @pl.when(i > 0)
    def _():
        mx_ref[...] = jnp.maximum(mx_ref[...], blkmax)

    cvec_ref[...] = jnp.concatenate(
        [jnp.zeros((1, 8), jnp.float32), mx_ref[:, 0:8]], axis=1)


def _prep(x, W, A):
    n, d_in = x.shape
    d = W.shape[1]
    nblk = n // ROW_BLK
    return pl.pallas_call(
        _prep_body,
        grid=(nblk,),
        in_specs=[
            pl.BlockSpec((ROW_BLK, d_in), lambda i: (i, 0)),
            pl.BlockSpec((d_in, d), lambda i: (0, 0)),
            pl.BlockSpec((d, 32), lambda i: (0, 0)),
        ],
        out_specs=[
            pl.BlockSpec((ROW_BLK, d), lambda i: (i, 0)),
            pl.BlockSpec((ROW_BLK, 16), lambda i: (i, 0)),
            pl.BlockSpec((ROW_BLK, 16), lambda i: (i, 0)),
            pl.BlockSpec((1, 16), lambda i: (0, 0)),
        ],
        out_shape=[
            jax.ShapeDtypeStruct((n, d), jnp.bfloat16),
            jax.ShapeDtypeStruct((n, 16), jnp.float32),
            jax.ShapeDtypeStruct((n, 16), jnp.float32),
            jax.ShapeDtypeStruct((1, 16), jnp.float32),
        ],
        scratch_shapes=[pltpu.VMEM((1, 16), jnp.float32)],
    )(x, W, A)


def _att_matrix(att_src, att_dst, heads, ch, perm):
    d = heads * ch
    A = jnp.zeros((d, 32), jnp.float32)
    hh = jnp.arange(heads)
    rows = (hh[:, None] * ch + jnp.arange(ch)[None, :]).reshape(-1)
    cols = jnp.repeat(hh, ch)
    asrc = att_src.reshape(-1)
    adst = att_dst.reshape(-1)
    A = A.at[rows, cols].set(asrc)
    A = A.at[rows, cols + 16].set(adst)
    A = A.at[rows, cols + 24].set(adst)
    return A[perm, :]


def _take16(v, idx):
    dn = lax.GatherDimensionNumbers(
        offset_dims=(), collapsed_slice_dims=(0,), start_index_map=(0,))
    return lax.gather(v, idx[:, None], dn, (1,),
                      mode=lax.GatherScatterMode.PROMISE_IN_BOUNDS)


def _make_edge_kernel(D, H):
    ROW = D + 16
    NVJ = D // 32
    mesh = plsc.VectorSubcoreMesh(core_axis_name="c", subcore_axis_name="s")

    @functools.partial(
        pl.kernel,
        out_type=jax.ShapeDtypeStruct((2, N_ACC, ROW), jnp.float32),
        mesh=mesh,
        scratch_types=[
            pltpu.VMEM((K_EDGE,), jnp.int32),
            pltpu.VMEM((K_EDGE,), jnp.int32),
            pltpu.VMEM((K_EDGE,), jnp.int32),
            pltpu.VMEM((K_EDGE,), jnp.int32),
            pltpu.VMEM((K_EDGE, D), jnp.bfloat16),
            pltpu.VMEM((K_EDGE, D), jnp.bfloat16),
            pltpu.VMEM((K_EDGE, 16), jnp.float32),
            pltpu.VMEM((K_EDGE, 16), jnp.float32),
            pltpu.VMEM((K_EDGE, 16), jnp.float32),
            pltpu.VMEM((K_EDGE, 16), jnp.float32),
            pltpu.VMEM((K_EDGE, ROW), jnp.float32),
            pltpu.VMEM((1, 16), jnp.float32),
            pltpu.VMEM_SHARED((N_ACC, ROW), jnp.float32),
            pltpu.SemaphoreType.DMA,
            pltpu.SemaphoreType.DMA,
            pltpu.SemaphoreType.DMA,
            pltpu.SemaphoreType.DMA,
            pltpu.SemaphoreType.DMA,
            pltpu.SemaphoreType.DMA,
        ],
        compiler_params=pltpu.CompilerParams(use_tc_tiling_on_sc=False,
                                             needs_layout_passes=False),
    )
    def edge_kernel(hbf_hbm, asrc_hbm, adst_hbm, cvec_hbm, sidx_hbm, didx_hbm,
                    zeros_hbm, out_hbm, si0, si1, di0, di1, rows0, rows1,
                    as0, as1, ad0, ad1, sbuf, cvec_v, acc_sh,
                    sh0, sh1, ss0, ss1, sa0, sa1):
        sidx = (si0, si1)
        didx = (di0, di1)
        rows = (rows0, rows1)
        asr = (as0, as1)
        ads = (ad0, ad1)
        sem_h = (sh0, sh1)
        sem_s = (ss0, ss1)
        sem_a = (sa0, sa1)
        cid = lax.axis_index("c")
        sid = lax.axis_index("s")
        wid = sid * 2 + cid
        rpc = 624
        pltpu.sync_copy(zeros_hbm.at[pl.ds(0, rpc)],
                        acc_sh.at[pl.ds(sid * rpc, rpc)])

        @pl.when(sid == 15)
        def _():
            pltpu.sync_copy(zeros_hbm.at[pl.ds(0, 16)],
                            acc_sh.at[pl.ds(9984, 16)])

        pltpu.sync_copy(cvec_hbm, cvec_v)
        plsc.subcore_barrier()
        cv = cvec_v[0]
        iot = lax.iota(jnp.int32, 16)
        shift_idx = (iot & 7) + 8

        base0 = wid * T_EDGE

        def issue_gather(st, b):
            base = base0 + st * K_EDGE
            pltpu.sync_copy(sidx_hbm.at[pl.ds(base, K_EDGE)], sidx[b])
            pltpu.sync_copy(didx_hbm.at[pl.ds(base, K_EDGE)], didx[b])
            pltpu.async_copy(hbf_hbm.at[sidx[b]], rows[b], sem_h[b])
            pltpu.async_copy(asrc_hbm.at[sidx[b]], asr[b], sem_s[b])
            pltpu.async_copy(adst_hbm.at[didx[b]], ads[b], sem_a[b])

        def wait_gather(st, b):
            pltpu.make_async_copy(hbf_hbm.at[sidx[b]], rows[b],
                                  sem_h[b]).wait()
            pltpu.make_async_copy(asrc_hbm.at[sidx[b]], asr[b],
                                  sem_s[b]).wait()
            pltpu.make_async_copy(adst_hbm.at[didx[b]], ads[b],
                                  sem_a[b]).wait()

        issue_gather(0, 0)

        @pl.loop(0, STEPS // 2)
        def _(it):
            for b in range(2):
                st = it * 2 + b
                rv = rows[b]

                @pl.when(st + 1 < STEPS)
                def _():
                    issue_gather(st + 1, 1 - b)

                wait_gather(st, b)

                @pl.loop(0, K_EDGE, unroll=2)
                def _(e):
                    v1 = asr[b][e]
                    v2 = ads[b][e] + cv
                    t = v1 + v2
                    al = jnp.where(t > 0, t, t * 0.2)
                    mm = _take16(al, shift_idx)
                    w16 = jnp.exp(al - mm)
                    sbuf[e, pl.ds(D, 16)] = w16
                    for j in range(NVJ):
                        hb = rv[e, pl.ds(32 * j, 32)]
                        p0, p1 = plsc.unpack(
                            hb, format=plsc.PackFormat.INTERLEAVED)
                        ha = 2 * j if H > 1 else 0
                        hc = 2 * j + 1 if H > 1 else 0
                        wa = _take16(w16, jnp.full((16,), ha, jnp.int32))
                        wb = _take16(w16, jnp.full((16,), hc, jnp.int32))
                        sbuf[e, pl.ds(32 * j, 16)] = p0 * wa
                        sbuf[e, pl.ds(32 * j + 16, 16)] = p1 * wb

                pltpu.sync_copy(sbuf, acc_sh.at[didx[b]], add=True)

        plsc.subcore_barrier()
        pltpu.sync_copy(acc_sh.at[pl.ds(sid * rpc, rpc)],
                        out_hbm.at[cid, pl.ds(sid * rpc, rpc)])

        @pl.when(sid == 15)
        def _():
            pltpu.sync_copy(acc_sh.at[pl.ds(9984, 16)],
                            out_hbm.at[cid, pl.ds(9984, 16)])

    return edge_kernel


_edge_kernel_l1 = _make_edge_kernel(128, 8)
_edge_kernel_l2 = _make_edge_kernel(64, 1)


def _epi1_body(acc_ref, r_ref, b_ref, w2_ref, a2_ref,
               hbf2_ref, asrc2_ref, adst2_ref, cvec2_ref, mx_ref):
    i = pl.program_id(0)
    comb = acc_ref[0] + acc_ref[1]
    num = comb[:, :128]
    den = jnp.dot(comb[:, 128:144], r_ref[...],
                  preferred_element_type=jnp.float32)
    pre = num / (den + 1e-16) + b_ref[...]
    x2 = jnp.where(pre > 0, pre, jnp.exp(pre) - 1.0)
    h2 = jnp.dot(x2, w2_ref[...], preferred_element_type=jnp.float32)
    ap2 = jnp.dot(h2, a2_ref[...], preferred_element_type=jnp.float32)
    hbf2_ref[...] = h2.astype(jnp.bfloat16)
    asrc2_ref[...] = ap2[:, 0:16]
    adst2_ref[...] = ap2[:, 16:32]
    blkmax = jnp.max(ap2[:, 0:16], axis=0, keepdims=True)

    @pl.when(i == 0)
    def _():
        mx_ref[...] = blkmax

    @pl.when(i > 0)
    def _():
        mx_ref[...] = jnp.maximum(mx_ref[...], blkmax)

    cvec2_ref[...] = jnp.concatenate(
        [jnp.zeros((1, 8), jnp.float32), mx_ref[:, 0:8]], axis=1)


def _epi1(acc1, R1, b1row, W2, A2):
    nblk = N_NODES // ROW_BLK
    return pl.pallas_call(
        _epi1_body,
        grid=(nblk,),
        in_specs=[
            pl.BlockSpec((2, ROW_BLK, 144), lambda i: (0, i, 0)),
            pl.BlockSpec((16, 128), lambda i: (0, 0)),
            pl.BlockSpec((1, 128), lambda i: (0, 0)),
            pl.BlockSpec((128, 64), lambda i: (0, 0)),
            pl.BlockSpec((64, 32), lambda i: (0, 0)),
        ],
        out_specs=[
            pl.BlockSpec((ROW_BLK, 64), lambda i: (i, 0)),
            pl.BlockSpec((ROW_BLK, 16), lambda i: (i, 0)),
            pl.BlockSpec((ROW_BLK, 16), lambda i: (i, 0)),
            pl.BlockSpec((1, 16), lambda i: (0, 0)),
        ],
        out_shape=[
            jax.ShapeDtypeStruct((N_NODES, 64), jnp.bfloat16),
            jax.ShapeDtypeStruct((N_NODES, 16), jnp.float32),
            jax.ShapeDtypeStruct((N_NODES, 16), jnp.float32),
            jax.ShapeDtypeStruct((1, 16), jnp.float32),
        ],
        scratch_shapes=[pltpu.VMEM((1, 16), jnp.float32)],
    )(acc1, R1, b1row, W2, A2)


def _epi2_body(acc_ref, r_ref, b_ref, out_ref):
    comb = acc_ref[0] + acc_ref[1]
    num = comb[:, :64]
    den = jnp.dot(comb[:, 64:80], r_ref[...],
                  preferred_element_type=jnp.float32)
    out_ref[...] = num / (den + 1e-16) + b_ref[...]


def _epi2(acc2, R2, b2row):
    nblk = N_NODES // ROW_BLK
    return pl.pallas_call(
        _epi2_body,
        grid=(nblk,),
        in_specs=[
            pl.BlockSpec((2, ROW_BLK, 80), lambda i: (0, i, 0)),
            pl.BlockSpec((16, 64), lambda i: (0, 0)),
            pl.BlockSpec((1, 64), lambda i: (0, 0)),
        ],
        out_specs=pl.BlockSpec((ROW_BLK, 64), lambda i: (i, 0)),
        out_shape=jax.ShapeDtypeStruct((N_NODES, 64), jnp.float32),
    )(acc2, R2, b2row)


def _bcast_matrix(heads, ch, d):
    R = jnp.zeros((16, d), jnp.float32)
    hh = jnp.repeat(jnp.arange(heads), ch)
    R = R.at[hh, jnp.arange(d)].set(1.0)
    return R


def _sentinel_asrc():
    r = jnp.zeros((1, 16), jnp.float32)
    return r.at[0, 0:8].set(-1e30)


def kernel(edge_index, node_emb, W1, att_src1, att_dst1, b1, W2, att_src2,
           att_dst2, b2):
    loops = jnp.arange(N_NODES, dtype=edge_index.dtype)
    ei = jnp.concatenate([edge_index, jnp.stack([loops, loops])], axis=1)
    ei = ei.astype(jnp.int32)
    pad = E_PAD - E_TOT
    sidx = jnp.concatenate([ei[0], jnp.full((pad,), N_NODES, jnp.int32)])
    didx = jnp.concatenate([ei[1], jnp.zeros((pad,), jnp.int32)])

    p1 = _perm(128, 8)
    p2 = _perm(64, 1)
    W1p = W1[:, p1]
    W2p = W2[:, p2]
    A1 = _att_matrix(att_src1, att_dst1, 8, 16, p1)
    A2 = _att_matrix(att_src2, att_dst2, 1, 64, p2)
    R1 = _bcast_matrix(8, 16, 128)
    R2 = _bcast_matrix(1, 64, 64)
    zeros1 = jnp.zeros((624, 144), jnp.float32)
    zeros2 = jnp.zeros((624, 80), jnp.float32)

    hbf1, asrc1, adst1, cvec1 = _prep(node_emb, W1p, A1)
    hbf1 = jnp.concatenate([hbf1, jnp.zeros((1, 128), jnp.bfloat16)])
    asrc1 = jnp.concatenate([asrc1, _sentinel_asrc()])
    acc1 = _edge_kernel_l1(hbf1, asrc1, adst1, cvec1, sidx, didx, zeros1)

    hbf2, asrc2, adst2, cvec2 = _epi1(acc1, R1, b1.reshape(1, 128), W2p, A2)
    hbf2 = jnp.concatenate([hbf2, jnp.zeros((1, 64), jnp.bfloat16)])
    asrc2 = jnp.concatenate([asrc2, _sentinel_asrc()])
    acc2 = _edge_kernel_l2(hbf2, asrc2, adst2, cvec2, sidx, didx, zeros2)

    return _epi2(acc2, R2, b2.reshape(1, 64))


# speedup vs baseline: 1.1988x; 1.1988x over previous
import functools

import jax
import jax.numpy as jnp
from jax import lax
from jax.experimental import pallas as pl
from jax.experimental.pallas import tpu as pltpu
from jax.experimental.pallas import tpu_sc as plsc

N_NODES = 10000
N_EDGES = 320000
E_TOT = N_EDGES + N_NODES
NW = 32
K_EDGE = 112
STEPS = 94
T_EDGE = STEPS * K_EDGE
E_PAD = NW * T_EDGE
N_ACC = 10000
ROW_BLK = 1000


def _prep_body(x_ref, w_ref, a_ref, hs_ref, adst_ref, cvec_ref, mx_ref):
    i = pl.program_id(0)
    d = w_ref.shape[1]
    h = jnp.dot(x_ref[...], w_ref[...], preferred_element_type=jnp.float32)
    ap = jnp.dot(h, a_ref[...], preferred_element_type=jnp.float32)
    hs_ref[:, :d] = h
    hs_ref[:, d:d + 16] = ap[:, 0:16]
    adst_ref[...] = ap[:, 16:32]
    blkmax = jnp.max(ap[:, 0:16], axis=0, keepdims=True)

    @pl.when(i == 0)
    def _():
        mx_ref[...] = blkmax

    @pl.when(i > 0)
    def _():
        mx_ref[...] = jnp.maximum(mx_ref[...], blkmax)

    cvec_ref[...] = jnp.concatenate(
        [jnp.zeros((1, 8), jnp.float32), mx_ref[:, 0:8]], axis=1)


def _prep(x, W, A):
    n, d_in = x.shape
    d = W.shape[1]
    nblk = n // ROW_BLK
    return pl.pallas_call(
        _prep_body,
        grid=(nblk,),
        in_specs=[
            pl.BlockSpec((ROW_BLK, d_in), lambda i: (i, 0)),
            pl.BlockSpec((d_in, d), lambda i: (0, 0)),
            pl.BlockSpec((d, 32), lambda i: (0, 0)),
        ],
        out_specs=[
            pl.BlockSpec((ROW_BLK, d + 16), lambda i: (i, 0)),
            pl.BlockSpec((ROW_BLK, 16), lambda i: (i, 0)),
            pl.BlockSpec((1, 16), lambda i: (0, 0)),
        ],
        out_shape=[
            jax.ShapeDtypeStruct((n, d + 16), jnp.float32),
            jax.ShapeDtypeStruct((n, 16), jnp.float32),
            jax.ShapeDtypeStruct((1, 16), jnp.float32),
        ],
        scratch_shapes=[pltpu.VMEM((1, 16), jnp.float32)],
    )(x, W, A)


def _att_matrix(att_src, att_dst, heads, ch):
    d = heads * ch
    A = jnp.zeros((d, 32), jnp.float32)
    hh = jnp.arange(heads)
    rows = (hh[:, None] * ch + jnp.arange(ch)[None, :]).reshape(-1)
    cols = jnp.repeat(hh, ch)
    asrc = att_src.reshape(-1)
    adst = att_dst.reshape(-1)
    A = A.at[rows, cols].set(asrc)
    A = A.at[rows, cols + 16].set(adst)
    A = A.at[rows, cols + 24].set(adst)
    return A


def _take16(v, idx):
    dn = lax.GatherDimensionNumbers(
        offset_dims=(), collapsed_slice_dims=(0,), start_index_map=(0,))
    return lax.gather(v, idx[:, None], dn, (1,),
                      mode=lax.GatherScatterMode.PROMISE_IN_BOUNDS)


def _make_edge_kernel(D, H):
    ROW = D + 16
    NV = D // 16
    mesh = plsc.VectorSubcoreMesh(core_axis_name="c", subcore_axis_name="s")

    @functools.partial(
        pl.kernel,
        out_type=jax.ShapeDtypeStruct((2, N_ACC, ROW), jnp.float32),
        mesh=mesh,
        scratch_types=[
            pltpu.VMEM((K_EDGE,), jnp.int32),
            pltpu.VMEM((K_EDGE,), jnp.int32),
            pltpu.VMEM((K_EDGE,), jnp.int32),
            pltpu.VMEM((K_EDGE,), jnp.int32),
            pltpu.VMEM((K_EDGE, ROW), jnp.float32),
            pltpu.VMEM((K_EDGE, ROW), jnp.float32),
            pltpu.VMEM((K_EDGE, 16), jnp.float32),
            pltpu.VMEM((K_EDGE, 16), jnp.float32),
            pltpu.VMEM((1, 16), jnp.float32),
            pltpu.VMEM_SHARED((N_ACC, ROW), jnp.float32),
            pltpu.SemaphoreType.DMA,
            pltpu.SemaphoreType.DMA,
            pltpu.SemaphoreType.DMA,
            pltpu.SemaphoreType.DMA,
        ],
        compiler_params=pltpu.CompilerParams(use_tc_tiling_on_sc=False),
    )
    def edge_kernel(hs_hbm, adst_hbm, cvec_hbm, sidx_hbm, didx_hbm, zeros_hbm,
                    out_hbm, si0, si1, di0, di1, rows0, rows1, ad0, ad1,
                    cvec_v, acc_sh, sr0, sr1, sa0, sa1):
        sidx = (si0, si1)
        didx = (di0, di1)
        rows = (rows0, rows1)
        ads = (ad0, ad1)
        sem_r = (sr0, sr1)
        sem_a = (sa0, sa1)
        cid = lax.axis_index("c")
        sid = lax.axis_index("s")
        wid = sid * 2 + cid
        rpc = 624
        pltpu.sync_copy(zeros_hbm.at[pl.ds(0, rpc)],
                        acc_sh.at[pl.ds(sid * rpc, rpc)])

        @pl.when(sid == 15)
        def _():
            pltpu.sync_copy(zeros_hbm.at[pl.ds(0, 16)],
                            acc_sh.at[pl.ds(9984, 16)])

        pltpu.sync_copy(cvec_hbm, cvec_v)
        plsc.subcore_barrier()
        cv = cvec_v[0]
        iot = lax.iota(jnp.int32, 16)
        shift_idx = (iot & 7) + 8

        base0 = wid * T_EDGE

        def issue_gather(st, b):
            base = base0 + st * K_EDGE
            pltpu.sync_copy(sidx_hbm.at[pl.ds(base, K_EDGE)], sidx[b])
            pltpu.sync_copy(didx_hbm.at[pl.ds(base, K_EDGE)], didx[b])
            pltpu.async_copy(hs_hbm.at[sidx[b]], rows[b], sem_r[b])
            pltpu.async_copy(adst_hbm.at[didx[b]], ads[b], sem_a[b])

        def wait_gather(st, b):
            pltpu.make_async_copy(hs_hbm.at[sidx[b]], rows[b],
                                  sem_r[b]).wait()
            pltpu.make_async_copy(adst_hbm.at[didx[b]], ads[b],
                                  sem_a[b]).wait()

        issue_gather(0, 0)

        @pl.loop(0, STEPS // 2)
        def _(it):
            for b in range(2):
                st = it * 2 + b
                rv = rows[b]

                @pl.when(st + 1 < STEPS)
                def _():
                    issue_gather(st + 1, 1 - b)

                wait_gather(st, b)

                @pl.loop(0, K_EDGE)
                def _(e):
                    v1 = rv[e, pl.ds(D, 16)]
                    v2 = ads[b][e] + cv
                    t = v1 + v2
                    al = jnp.where(t > 0, t, t * 0.2)
                    mm = _take16(al, shift_idx)
                    w16 = jnp.exp(al - mm)
                    rv[e, pl.ds(D, 16)] = w16
                    for j in range(NV):
                        hj = j if H > 1 else 0
                        ws = _take16(w16, jnp.full((16,), hj, jnp.int32))
                        rv[e, pl.ds(j * 16, 16)] = (
                            rv[e, pl.ds(j * 16, 16)] * ws)

                pltpu.sync_copy(rv, acc_sh.at[didx[b]], add=True)

        plsc.subcore_barrier()
        pltpu.sync_copy(acc_sh.at[pl.ds(sid * rpc, rpc)],
                        out_hbm.at[cid, pl.ds(sid * rpc, rpc)])

        @pl.when(sid == 15)
        def _():
            pltpu.sync_copy(acc_sh.at[pl.ds(9984, 16)],
                            out_hbm.at[cid, pl.ds(9984, 16)])

    return edge_kernel


_edge_kernel_l1 = _make_edge_kernel(128, 8)
_edge_kernel_l2 = _make_edge_kernel(64, 1)


def _epi1_body(acc_ref, r_ref, b_ref, w2_ref, a2_ref,
               hs2_ref, adst2_ref, cvec2_ref, mx_ref):
    i = pl.program_id(0)
    comb = acc_ref[0] + acc_ref[1]
    num = comb[:, :128]
    den = jnp.dot(comb[:, 128:144], r_ref[...],
                  preferred_element_type=jnp.float32)
    pre = num / (den + 1e-16) + b_ref[...]
    x2 = jnp.where(pre > 0, pre, jnp.exp(pre) - 1.0)
    h2 = jnp.dot(x2, w2_ref[...], preferred_element_type=jnp.float32)
    ap2 = jnp.dot(h2, a2_ref[...], preferred_element_type=jnp.float32)
    hs2_ref[:, :64] = h2
    hs2_ref[:, 64:80] = ap2[:, 0:16]
    adst2_ref[...] = ap2[:, 16:32]
    blkmax = jnp.max(ap2[:, 0:16], axis=0, keepdims=True)

    @pl.when(i == 0)
    def _():
        mx_ref[...] = blkmax

    @pl.when(i > 0)
    def _():
        mx_ref[...] = jnp.maximum(mx_ref[...], blkmax)

    cvec2_ref[...] = jnp.concatenate(
        [jnp.zeros((1, 8), jnp.float32), mx_ref[:, 0:8]], axis=1)


def _epi1(acc1, R1, b1row, W2, A2):
    nblk = N_NODES // ROW_BLK
    return pl.pallas_call(
        _epi1_body,
        grid=(nblk,),
        in_specs=[
            pl.BlockSpec((2, ROW_BLK, 144), lambda i: (0, i, 0)),
            pl.BlockSpec((16, 128), lambda i: (0, 0)),
            pl.BlockSpec((1, 128), lambda i: (0, 0)),
            pl.BlockSpec((128, 64), lambda i: (0, 0)),
            pl.BlockSpec((64, 32), lambda i: (0, 0)),
        ],
        out_specs=[
            pl.BlockSpec((ROW_BLK, 80), lambda i: (i, 0)),
            pl.BlockSpec((ROW_BLK, 16), lambda i: (i, 0)),
            pl.BlockSpec((1, 16), lambda i: (0, 0)),
        ],
        out_shape=[
            jax.ShapeDtypeStruct((N_NODES, 80), jnp.float32),
            jax.ShapeDtypeStruct((N_NODES, 16), jnp.float32),
            jax.ShapeDtypeStruct((1, 16), jnp.float32),
        ],
        scratch_shapes=[pltpu.VMEM((1, 16), jnp.float32)],
    )(acc1, R1, b1row, W2, A2)


def _epi2_body(acc_ref, r_ref, b_ref, out_ref):
    comb = acc_ref[0] + acc_ref[1]
    num = comb[:, :64]
    den = jnp.dot(comb[:, 64:80], r_ref[...],
                  preferred_element_type=jnp.float32)
    out_ref[...] = num / (den + 1e-16) + b_ref[...]


def _epi2(acc2, R2, b2row):
    nblk = N_NODES // ROW_BLK
    return pl.pallas_call(
        _epi2_body,
        grid=(nblk,),
        in_specs=[
            pl.BlockSpec((2, ROW_BLK, 80), lambda i: (0, i, 0)),
            pl.BlockSpec((16, 64), lambda i: (0, 0)),
            pl.BlockSpec((1, 64), lambda i: (0, 0)),
        ],
        out_specs=pl.BlockSpec((ROW_BLK, 64), lambda i: (i, 0)),
        out_shape=jax.ShapeDtypeStruct((N_NODES, 64), jnp.float32),
    )(acc2, R2, b2row)


def _bcast_matrix(heads, ch, d):
    R = jnp.zeros((16, d), jnp.float32)
    hh = jnp.repeat(jnp.arange(heads), ch)
    R = R.at[hh, jnp.arange(d)].set(1.0)
    return R


def _sentinel_row(d):
    r = jnp.zeros((1, d + 16), jnp.float32)
    return r.at[0, d:d + 8].set(-1e30)


def kernel(edge_index, node_emb, W1, att_src1, att_dst1, b1, W2, att_src2,
           att_dst2, b2):
    loops = jnp.arange(N_NODES, dtype=edge_index.dtype)
    ei = jnp.concatenate([edge_index, jnp.stack([loops, loops])], axis=1)
    ei = ei.astype(jnp.int32)
    pad = E_PAD - E_TOT
    sidx = jnp.concatenate([ei[0], jnp.full((pad,), N_NODES, jnp.int32)])
    didx = jnp.concatenate([ei[1], jnp.zeros((pad,), jnp.int32)])

    A1 = _att_matrix(att_src1, att_dst1, 8, 16)
    A2 = _att_matrix(att_src2, att_dst2, 1, 64)
    R1 = _bcast_matrix(8, 16, 128)
    R2 = _bcast_matrix(1, 64, 64)
    zeros1 = jnp.zeros((624, 144), jnp.float32)
    zeros2 = jnp.zeros((624, 80), jnp.float32)

    hs1, adst1, cvec1 = _prep(node_emb, W1, A1)
    hs1 = jnp.concatenate([hs1, _sentinel_row(128)])
    acc1 = _edge_kernel_l1(hs1, adst1, cvec1, sidx, didx, zeros1)

    hs2, adst2, cvec2 = _epi1(acc1, R1, b1.reshape(1, 128), W2, A2)
    hs2 = jnp.concatenate([hs2, _sentinel_row(64)])
    acc2 = _edge_kernel_l2(hs2, adst2, cvec2, sidx, didx, zeros2)

    return _epi2(acc2, R2, b2.reshape(1, 64))


# speedup vs baseline: 1.3166x; 1.0983x over previous
import functools

import jax
import jax.numpy as jnp
from jax import lax
from jax.experimental import pallas as pl
from jax.experimental.pallas import tpu as pltpu
from jax.experimental.pallas import tpu_sc as plsc

N_NODES = 10000
N_EDGES = 320000
E_TOT = N_EDGES + N_NODES
NW = 32
K_EDGE = 120
STEPS = 86
T_EDGE = STEPS * K_EDGE
E_PAD = NW * T_EDGE
N_ACC = 10000
ROW_BLK = 1000


def _prep_body(x_ref, w_ref, a_ref, hs_ref, adst_ref, cvec_ref, mx_ref):
    i = pl.program_id(0)
    d = w_ref.shape[1]
    h = jnp.dot(x_ref[...], w_ref[...], preferred_element_type=jnp.float32)
    ap = jnp.dot(h, a_ref[...], preferred_element_type=jnp.float32)
    hs_ref[:, :d] = h
    hs_ref[:, d:d + 16] = ap[:, 0:16]
    adst_ref[...] = ap[:, 16:32]
    blkmax = jnp.max(ap[:, 0:16], axis=0, keepdims=True)

    @pl.when(i == 0)
    def _():
        mx_ref[...] = blkmax

    @pl.when(i > 0)
    def _():
        mx_ref[...] = jnp.maximum(mx_ref[...], blkmax)

    cvec_ref[...] = jnp.concatenate(
        [jnp.zeros((1, 8), jnp.float32), mx_ref[:, 0:8]], axis=1)


def _prep(x, W, A):
    n, d_in = x.shape
    d = W.shape[1]
    nblk = n // ROW_BLK
    return pl.pallas_call(
        _prep_body,
        grid=(nblk,),
        in_specs=[
            pl.BlockSpec((ROW_BLK, d_in), lambda i: (i, 0)),
            pl.BlockSpec((d_in, d), lambda i: (0, 0)),
            pl.BlockSpec((d, 32), lambda i: (0, 0)),
        ],
        out_specs=[
            pl.BlockSpec((ROW_BLK, d + 16), lambda i: (i, 0)),
            pl.BlockSpec((ROW_BLK, 16), lambda i: (i, 0)),
            pl.BlockSpec((1, 16), lambda i: (0, 0)),
        ],
        out_shape=[
            jax.ShapeDtypeStruct((n, d + 16), jnp.float32),
            jax.ShapeDtypeStruct((n, 16), jnp.float32),
            jax.ShapeDtypeStruct((1, 16), jnp.float32),
        ],
        scratch_shapes=[pltpu.VMEM((1, 16), jnp.float32)],
    )(x, W, A)


def _att_matrix(att_src, att_dst, heads, ch):
    d = heads * ch
    A = jnp.zeros((d, 32), jnp.float32)
    hh = jnp.arange(heads)
    rows = (hh[:, None] * ch + jnp.arange(ch)[None, :]).reshape(-1)
    cols = jnp.repeat(hh, ch)
    asrc = att_src.reshape(-1)
    adst = att_dst.reshape(-1)
    A = A.at[rows, cols].set(asrc)
    A = A.at[rows, cols + 16].set(adst)
    A = A.at[rows, cols + 24].set(adst)
    return A


def _take16(v, idx):
    dn = lax.GatherDimensionNumbers(
        offset_dims=(), collapsed_slice_dims=(0,), start_index_map=(0,))
    return lax.gather(v, idx[:, None], dn, (1,),
                      mode=lax.GatherScatterMode.PROMISE_IN_BOUNDS)


def _make_edge_kernel(D, H):
    ROW = D + 16
    NV = D // 16
    mesh = plsc.VectorSubcoreMesh(core_axis_name="c", subcore_axis_name="s")

    @functools.partial(
        pl.kernel,
        out_type=jax.ShapeDtypeStruct((2, N_ACC, ROW), jnp.float32),
        mesh=mesh,
        scratch_types=[
            pltpu.VMEM((K_EDGE,), jnp.int32),
            pltpu.VMEM((K_EDGE,), jnp.int32),
            pltpu.VMEM((K_EDGE,), jnp.int32),
            pltpu.VMEM((K_EDGE,), jnp.int32),
            pltpu.VMEM((K_EDGE, ROW), jnp.float32),
            pltpu.VMEM((K_EDGE, ROW), jnp.float32),
            pltpu.VMEM((K_EDGE, 16), jnp.float32),
            pltpu.VMEM((K_EDGE, 16), jnp.float32),
            pltpu.VMEM((1, 16), jnp.float32),
            pltpu.VMEM_SHARED((N_ACC, ROW), jnp.float32),
            pltpu.SemaphoreType.DMA,
            pltpu.SemaphoreType.DMA,
            pltpu.SemaphoreType.DMA,
            pltpu.SemaphoreType.DMA,
        ],
        compiler_params=pltpu.CompilerParams(use_tc_tiling_on_sc=False),
    )
    def edge_kernel(hs_hbm, adst_hbm, cvec_hbm, sidx_hbm, didx_hbm, zeros_hbm,
                    out_hbm, si0, si1, di0, di1, rows0, rows1, ad0, ad1,
                    cvec_v, acc_sh, sr0, sr1, sa0, sa1):
        sidx = (si0, si1)
        didx = (di0, di1)
        rows = (rows0, rows1)
        ads = (ad0, ad1)
        sem_r = (sr0, sr1)
        sem_a = (sa0, sa1)
        cid = lax.axis_index("c")
        sid = lax.axis_index("s")
        wid = sid * 2 + cid
        rpc = 624
        pltpu.sync_copy(zeros_hbm.at[pl.ds(0, rpc)],
                        acc_sh.at[pl.ds(sid * rpc, rpc)])

        @pl.when(sid == 15)
        def _():
            pltpu.sync_copy(zeros_hbm.at[pl.ds(0, 16)],
                            acc_sh.at[pl.ds(9984, 16)])

        pltpu.sync_copy(cvec_hbm, cvec_v)
        plsc.subcore_barrier()
        cv = cvec_v[0]
        iot = lax.iota(jnp.int32, 16)
        shift_idx = (iot & 7) + 8

        base0 = wid * T_EDGE

        def issue_gather(st, b):
            base = base0 + st * K_EDGE
            pltpu.sync_copy(sidx_hbm.at[pl.ds(base, K_EDGE)], sidx[b])
            pltpu.sync_copy(didx_hbm.at[pl.ds(base, K_EDGE)], didx[b])
            pltpu.async_copy(hs_hbm.at[sidx[b]], rows[b], sem_r[b])
            pltpu.async_copy(adst_hbm.at[didx[b]], ads[b], sem_a[b])

        def wait_gather(st, b):
            pltpu.make_async_copy(hs_hbm.at[sidx[b]], rows[b],
                                  sem_r[b]).wait()
            pltpu.make_async_copy(adst_hbm.at[didx[b]], ads[b],
                                  sem_a[b]).wait()

        issue_gather(0, 0)

        @pl.loop(0, STEPS // 2)
        def _(it):
            for b in range(2):
                st = it * 2 + b
                rv = rows[b]

                @pl.when(st + 1 < STEPS)
                def _():
                    issue_gather(st + 1, 1 - b)

                wait_gather(st, b)

                @pl.loop(0, K_EDGE)
                def _(e):
                    v1 = rv[e, pl.ds(D, 16)]
                    v2 = ads[b][e] + cv
                    t = v1 + v2
                    al = jnp.where(t > 0, t, t * 0.2)
                    mm = _take16(al, shift_idx)
                    w16 = jnp.exp(al - mm)
                    rv[e, pl.ds(D, 16)] = w16
                    for j in range(NV):
                        hj = j if H > 1 else 0
                        ws = _take16(w16, jnp.full((16,), hj, jnp.int32))
                        rv[e, pl.ds(j * 16, 16)] = (
                            rv[e, pl.ds(j * 16, 16)] * ws)

                pltpu.sync_copy(rv, acc_sh.at[didx[b]], add=True)

        plsc.subcore_barrier()
        pltpu.sync_copy(acc_sh.at[pl.ds(sid * rpc, rpc)],
                        out_hbm.at[cid, pl.ds(sid * rpc, rpc)])

        @pl.when(sid == 15)
        def _():
            pltpu.sync_copy(acc_sh.at[pl.ds(9984, 16)],
                            out_hbm.at[cid, pl.ds(9984, 16)])

    return edge_kernel


_edge_kernel_l1 = _make_edge_kernel(128, 8)
_edge_kernel_l2 = _make_edge_kernel(64, 1)


def _epi1_body(acc_ref, r_ref, b_ref, w2_ref, a2_ref,
               hs2_ref, adst2_ref, cvec2_ref, mx_ref):
    i = pl.program_id(0)
    comb = acc_ref[0] + acc_ref[1]
    num = comb[:, :128]
    den = jnp.dot(comb[:, 128:144], r_ref[...],
                  preferred_element_type=jnp.float32)
    pre = num / (den + 1e-16) + b_ref[...]
    x2 = jnp.where(pre > 0, pre, jnp.exp(pre) - 1.0)
    h2 = jnp.dot(x2, w2_ref[...], preferred_element_type=jnp.float32)
    ap2 = jnp.dot(h2, a2_ref[...], preferred_element_type=jnp.float32)
    hs2_ref[:, :64] = h2
    hs2_ref[:, 64:80] = ap2[:, 0:16]
    adst2_ref[...] = ap2[:, 16:32]
    blkmax = jnp.max(ap2[:, 0:16], axis=0, keepdims=True)

    @pl.when(i == 0)
    def _():
        mx_ref[...] = blkmax

    @pl.when(i > 0)
    def _():
        mx_ref[...] = jnp.maximum(mx_ref[...], blkmax)

    cvec2_ref[...] = jnp.concatenate(
        [jnp.zeros((1, 8), jnp.float32), mx_ref[:, 0:8]], axis=1)


def _epi1(acc1, R1, b1row, W2, A2):
    nblk = N_NODES // ROW_BLK
    return pl.pallas_call(
        _epi1_body,
        grid=(nblk,),
        in_specs=[
            pl.BlockSpec((2, ROW_BLK, 144), lambda i: (0, i, 0)),
            pl.BlockSpec((16, 128), lambda i: (0, 0)),
            pl.BlockSpec((1, 128), lambda i: (0, 0)),
            pl.BlockSpec((128, 64), lambda i: (0, 0)),
            pl.BlockSpec((64, 32), lambda i: (0, 0)),
        ],
        out_specs=[
            pl.BlockSpec((ROW_BLK, 80), lambda i: (i, 0)),
            pl.BlockSpec((ROW_BLK, 16), lambda i: (i, 0)),
            pl.BlockSpec((1, 16), lambda i: (0, 0)),
        ],
        out_shape=[
            jax.ShapeDtypeStruct((N_NODES, 80), jnp.float32),
            jax.ShapeDtypeStruct((N_NODES, 16), jnp.float32),
            jax.ShapeDtypeStruct((1, 16), jnp.float32),
        ],
        scratch_shapes=[pltpu.VMEM((1, 16), jnp.float32)],
    )(acc1, R1, b1row, W2, A2)


def _epi2_body(acc_ref, r_ref, b_ref, out_ref):
    comb = acc_ref[0] + acc_ref[1]
    num = comb[:, :64]
    den = jnp.dot(comb[:, 64:80], r_ref[...],
                  preferred_element_type=jnp.float32)
    out_ref[...] = num / (den + 1e-16) + b_ref[...]


def _epi2(acc2, R2, b2row):
    nblk = N_NODES // ROW_BLK
    return pl.pallas_call(
        _epi2_body,
        grid=(nblk,),
        in_specs=[
            pl.BlockSpec((2, ROW_BLK, 80), lambda i: (0, i, 0)),
            pl.BlockSpec((16, 64), lambda i: (0, 0)),
            pl.BlockSpec((1, 64), lambda i: (0, 0)),
        ],
        out_specs=pl.BlockSpec((ROW_BLK, 64), lambda i: (i, 0)),
        out_shape=jax.ShapeDtypeStruct((N_NODES, 64), jnp.float32),
    )(acc2, R2, b2row)


def _bcast_matrix(heads, ch, d):
    R = jnp.zeros((16, d), jnp.float32)
    hh = jnp.repeat(jnp.arange(heads), ch)
    R = R.at[hh, jnp.arange(d)].set(1.0)
    return R


def _sentinel_row(d):
    r = jnp.zeros((1, d + 16), jnp.float32)
    return r.at[0, d:d + 8].set(-1e30)


def kernel(edge_index, node_emb, W1, att_src1, att_dst1, b1, W2, att_src2,
           att_dst2, b2):
    loops = jnp.arange(N_NODES, dtype=edge_index.dtype)
    ei = jnp.concatenate([edge_index, jnp.stack([loops, loops])], axis=1)
    ei = ei.astype(jnp.int32)
    pad = E_PAD - E_TOT
    sidx = jnp.concatenate([ei[0], jnp.full((pad,), N_NODES, jnp.int32)])
    didx = jnp.concatenate([ei[1], jnp.zeros((pad,), jnp.int32)])

    A1 = _att_matrix(att_src1, att_dst1, 8, 16)
    A2 = _att_matrix(att_src2, att_dst2, 1, 64)
    R1 = _bcast_matrix(8, 16, 128)
    R2 = _bcast_matrix(1, 64, 64)
    zeros1 = jnp.zeros((624, 144), jnp.float32)
    zeros2 = jnp.zeros((624, 80), jnp.float32)

    hs1, adst1, cvec1 = _prep(node_emb, W1, A1)
    hs1 = jnp.concatenate([hs1, _sentinel_row(128)])
    acc1 = _edge_kernel_l1(hs1, adst1, cvec1, sidx, didx, zeros1)

    hs2, adst2, cvec2 = _epi1(acc1, R1, b1.reshape(1, 128), W2, A2)
    hs2 = jnp.concatenate([hs2, _sentinel_row(64)])
    acc2 = _edge_kernel_l2(hs2, adst2, cvec2, sidx, didx, zeros2)

    return _epi2(acc2, R2, b2.reshape(1, 64))
